# baseline (speedup 1.0000x reference)
import functools

import jax
import jax.numpy as jnp
from jax import lax
from jax.experimental import pallas as pl
from jax.experimental.pallas import tpu as pltpu
from jax.experimental.pallas import tpu_sc as plsc

F32 = jnp.float32
NC = 2
NS = 16
NW = NC * NS
CHUNK = 128
DEGW = 16


def _deg_body(n_pad, cpw, dst_hbm, zeros_hbm, out_hbm,
              hist_v, idx_v, buf_v, col_v, shared):
    rows_pt = n_pad // NS
    cid = lax.axis_index("c")
    sid = lax.axis_index("s")
    wid = cid * NS + sid
    r0 = sid * rows_pt
    pltpu.sync_copy(zeros_hbm, hist_v)
    base0 = wid * (cpw * CHUNK)
    ones16 = jnp.ones((16,), F32)

    def body(c, carry):
        b = base0 + c * CHUNK
        pltpu.sync_copy(dst_hbm.at[pl.ds(b, CHUNK)], idx_v)
        for j in range(CHUNK // 16):
            vec = idx_v[pl.ds(16 * j, 16)]
            plsc.addupdate_scatter(hist_v, [vec], ones16)
        return carry

    lax.fori_loop(0, cpw, body, 0)

    pltpu.sync_copy(hist_v, shared.at[sid])
    plsc.subcore_barrier()
    pltpu.sync_copy(shared.at[:, pl.ds(r0, rows_pt)], buf_v)

    def red(k, carry):
        acc = buf_v[0, pl.ds(16 * k, 16)]
        for t in range(1, NS):
            acc = acc + buf_v[t, pl.ds(16 * k, 16)]
        col_v[pl.ds(16 * k, 16)] = acc
        return carry

    lax.fori_loop(0, rows_pt // 16, red, 0)
    pltpu.sync_copy(col_v, out_hbm.at[cid, pl.ds(r0, rows_pt)])


def _spmm_body(n_pad, cpw, sd_hbm, tab_hbm, zeros_hbm, out_hbm,
               acc_sh, sd0_v, sd1_v, rows0_v, rows1_v, sem0, sem1):
    rows_pt = n_pad // NS
    cid = lax.axis_index("c")
    sid = lax.axis_index("s")
    wid = cid * NS + sid
    r0 = sid * rows_pt
    pltpu.sync_copy(zeros_hbm.at[pl.ds(r0, rows_pt)], acc_sh.at[pl.ds(r0, rows_pt)])
    plsc.subcore_barrier()

    g0 = wid * cpw

    def body(c, carry):
        pltpu.sync_copy(sd_hbm.at[g0 + c], sd0_v)
        pltpu.async_copy(tab_hbm.at[sd0_v.at[0]], rows0_v, sem0).wait()
        pltpu.sync_copy(rows0_v, acc_sh.at[sd0_v.at[1]], add=True)
        return carry

    lax.fori_loop(0, cpw, body, 0)
    plsc.subcore_barrier()
    pltpu.sync_copy(acc_sh.at[pl.ds(r0, rows_pt)],
                    out_hbm.at[cid, pl.ds(r0, rows_pt)])


def _make_deg_kernel(n_pad, cpw):
    mesh = plsc.VectorSubcoreMesh(core_axis_name="c", subcore_axis_name="s",
                                  num_cores=NC, num_subcores=NS)
    return pl.kernel(
        functools.partial(_deg_body, n_pad, cpw),
        out_type=jax.ShapeDtypeStruct((NC, n_pad), F32),
        mesh=mesh,
        compiler_params=pltpu.CompilerParams(needs_layout_passes=False),
        scratch_types=[
            pltpu.VMEM((n_pad,), F32),
            pltpu.VMEM((CHUNK,), jnp.int32),
            pltpu.VMEM((NS, n_pad // NS), F32),
            pltpu.VMEM((n_pad // NS,), F32),
            pltpu.VMEM_SHARED((NS, n_pad), F32),
        ],
    )


def _make_spmm_kernel(n_pad, cpw, f):
    mesh = plsc.VectorSubcoreMesh(core_axis_name="c", subcore_axis_name="s",
                                  num_cores=NC, num_subcores=NS)
    return pl.kernel(
        functools.partial(_spmm_body, n_pad, cpw),
        out_type=jax.ShapeDtypeStruct((NC, n_pad, f), F32),
        mesh=mesh,
        scratch_types=[
            pltpu.VMEM_SHARED((n_pad, f), F32),
            pltpu.VMEM((2, CHUNK), jnp.int32),
            pltpu.VMEM((2, CHUNK), jnp.int32),
            pltpu.VMEM((CHUNK, f), F32),
            pltpu.VMEM((CHUNK, f), F32),
            pltpu.SemaphoreType.DMA,
            pltpu.SemaphoreType.DMA,
        ],
    )


def _tc_scale_body(p0, p1, x, dinv_ref, xp_ref):
    deg = p0[...] + p1[...] + 1.0
    dinv = lax.rsqrt(deg)
    dinv_ref[...] = dinv
    xp_ref[...] = x[...] * dinv


def _tc_mlp_body(p0, p1, xp, dinv, w1, b1, w2, yp_ref):
    dv = dinv[...]
    agg = dv * (p0[...] + p1[...] + xp[...])
    h = jnp.maximum(jnp.dot(agg, w1[...], preferred_element_type=F32,
                            precision=lax.Precision.HIGHEST) + b1[...], 0.0)
    yp_ref[...] = jnp.dot(h, w2[...], preferred_element_type=F32,
                          precision=lax.Precision.HIGHEST) * dv


def _tc_out_body(q0, q1, yp, dinv, b2, out_ref):
    dv = dinv[...]
    out_ref[...] = jnp.maximum(dv * (q0[...] + q1[...] + yp[...]) + b2[...], 0.0)


def _row_spec(rb, w):
    return pl.BlockSpec((rb, w), lambda i: (i, 0))


def _full_spec(shape):
    return pl.BlockSpec(shape, lambda i: tuple(0 for _ in shape))


def _tc_scale(p0, p1, x, rb=2048):
    n_pad, f = x.shape
    return pl.pallas_call(
        _tc_scale_body,
        grid=(n_pad // rb,),
        in_specs=[_row_spec(rb, 1), _row_spec(rb, 1), _row_spec(rb, f)],
        out_specs=[_row_spec(rb, 1), _row_spec(rb, f)],
        out_shape=[jax.ShapeDtypeStruct((n_pad, 1), F32),
                   jax.ShapeDtypeStruct((n_pad, f), F32)],
    )(p0, p1, x)


def _tc_mlp(p0, p1, xp, dinv, w1, b1, w2, rb=2048):
    n_pad, f = xp.shape
    h = w1.shape[1]
    return pl.pallas_call(
        _tc_mlp_body,
        grid=(n_pad // rb,),
        in_specs=[_row_spec(rb, f), _row_spec(rb, f), _row_spec(rb, f),
                  _row_spec(rb, 1), _full_spec((f, h)), _full_spec((1, h)),
                  _full_spec((h, f))],
        out_specs=_row_spec(rb, f),
        out_shape=jax.ShapeDtypeStruct((n_pad, f), F32),
    )(p0, p1, xp, dinv, w1, b1, w2)


def _tc_out(q0, q1, yp, dinv, b2, rb=2048):
    n_pad, f = yp.shape
    return pl.pallas_call(
        _tc_out_body,
        grid=(n_pad // rb,),
        in_specs=[_row_spec(rb, f), _row_spec(rb, f), _row_spec(rb, f),
                  _row_spec(rb, 1), _full_spec((1, f))],
        out_specs=_row_spec(rb, f),
        out_shape=jax.ShapeDtypeStruct((n_pad, f), F32),
    )(q0, q1, yp, dinv, b2)


def kernel(x, edge_index, W1, b1, W2, b2):
    n, f = x.shape
    e = edge_index.shape[1]
    n_pad = ((n + 1 + 2047) // 2048) * 2048
    cpw = -(-e // (NW * CHUNK))
    cpw += cpw % 2
    e_pad = NW * cpw * CHUNK

    pad = jnp.full((e_pad - e,), n, jnp.int32)
    src = jnp.concatenate([edge_index[0].astype(jnp.int32), pad])
    dst = jnp.concatenate([edge_index[1].astype(jnp.int32), pad])
    sd = jnp.stack([src.reshape(-1, CHUNK), dst.reshape(-1, CHUNK)], axis=1)
    x_pad = jnp.zeros((n_pad, f), F32).at[:n].set(x)
    zeros_row = jnp.zeros((n_pad,), F32)
    zeros_tab = jnp.zeros((n_pad, f), F32)

    deg_parts = _make_deg_kernel(n_pad, cpw)(dst, zeros_row)
    deg_cols = deg_parts.reshape(NC, n_pad, 1)
    dinv, xp = _tc_scale(deg_cols[0], deg_cols[1], x_pad)

    spmm = _make_spmm_kernel(n_pad, cpw, f)
    s1 = spmm(sd, xp, zeros_tab)
    yp = _tc_mlp(s1[0], s1[1], xp, dinv, W1, b1.reshape(1, -1), W2)
    s2 = spmm(sd, yp, zeros_tab)
    out = _tc_out(s2[0], s2[1], yp, dinv, b2.reshape(1, -1))
    return out[:n]


# speedup vs baseline: 1.3779x; 1.3779x over previous
import functools

import jax
import jax.numpy as jnp
from jax import lax
from jax.experimental import pallas as pl
from jax.experimental.pallas import tpu as pltpu
from jax.experimental.pallas import tpu_sc as plsc

F32 = jnp.float32
NC = 2
NS = 16
NW = NC * NS
CHUNK = 128
DEGW = 16


def _deg_body(n_pad, cpw, dst_hbm, zeros_hbm, out_hbm,
              hist_v, idx_v, buf_v, col_v, shared):
    rows_pt = n_pad // NS
    cid = lax.axis_index("c")
    sid = lax.axis_index("s")
    wid = cid * NS + sid
    r0 = sid * rows_pt
    pltpu.sync_copy(zeros_hbm, hist_v)
    base0 = wid * (cpw * CHUNK)
    ones16 = jnp.ones((16,), F32)

    def body(c, carry):
        b = base0 + c * CHUNK
        pltpu.sync_copy(dst_hbm.at[pl.ds(b, CHUNK)], idx_v)
        for j in range(CHUNK // 16):
            vec = idx_v[pl.ds(16 * j, 16)]
            plsc.addupdate_scatter(hist_v, [vec], ones16)
        return carry

    lax.fori_loop(0, cpw, body, 0)

    pltpu.sync_copy(hist_v, shared.at[sid])
    plsc.subcore_barrier()
    pltpu.sync_copy(shared.at[:, pl.ds(r0, rows_pt)], buf_v)

    def red(k, carry):
        acc = buf_v[0, pl.ds(16 * k, 16)]
        for t in range(1, NS):
            acc = acc + buf_v[t, pl.ds(16 * k, 16)]
        col_v[pl.ds(16 * k, 16)] = acc
        return carry

    lax.fori_loop(0, rows_pt // 16, red, 0)
    pltpu.sync_copy(col_v, out_hbm.at[cid, pl.ds(r0, rows_pt)])


def _spmm_body(n_pad, cpw, src_hbm, dst_hbm, tab_hbm, zeros_hbm, out_hbm,
               acc_sh, sidx_v, didx_v, rows_v, sem):
    rows_pt = n_pad // NS
    cid = lax.axis_index("c")
    sid = lax.axis_index("s")
    wid = cid * NS + sid
    r0 = sid * rows_pt
    pltpu.sync_copy(zeros_hbm.at[pl.ds(r0, rows_pt)], acc_sh.at[pl.ds(r0, rows_pt)])
    plsc.subcore_barrier()

    base0 = wid * (cpw * CHUNK)

    def body(c, carry):
        b = base0 + c * CHUNK
        pltpu.sync_copy(src_hbm.at[pl.ds(b, CHUNK)], sidx_v)
        pltpu.sync_copy(dst_hbm.at[pl.ds(b, CHUNK)], didx_v)
        pltpu.async_copy(tab_hbm.at[sidx_v], rows_v, sem).wait()
        pltpu.sync_copy(rows_v, acc_sh.at[didx_v], add=True)
        return carry

    lax.fori_loop(0, cpw, body, 0)
    plsc.subcore_barrier()
    pltpu.sync_copy(acc_sh.at[pl.ds(r0, rows_pt)],
                    out_hbm.at[cid, pl.ds(r0, rows_pt)])


def _make_deg_kernel(n_pad, cpw):
    mesh = plsc.VectorSubcoreMesh(core_axis_name="c", subcore_axis_name="s",
                                  num_cores=NC, num_subcores=NS)
    return pl.kernel(
        functools.partial(_deg_body, n_pad, cpw),
        out_type=jax.ShapeDtypeStruct((NC, n_pad), F32),
        mesh=mesh,
        compiler_params=pltpu.CompilerParams(needs_layout_passes=False),
        scratch_types=[
            pltpu.VMEM((n_pad,), F32),
            pltpu.VMEM((CHUNK,), jnp.int32),
            pltpu.VMEM((NS, n_pad // NS), F32),
            pltpu.VMEM((n_pad // NS,), F32),
            pltpu.VMEM_SHARED((NS, n_pad), F32),
        ],
    )


def _make_spmm_kernel(n_pad, cpw, f):
    mesh = plsc.VectorSubcoreMesh(core_axis_name="c", subcore_axis_name="s",
                                  num_cores=NC, num_subcores=NS)
    return pl.kernel(
        functools.partial(_spmm_body, n_pad, cpw),
        out_type=jax.ShapeDtypeStruct((NC, n_pad, f), F32),
        mesh=mesh,
        scratch_types=[
            pltpu.VMEM_SHARED((n_pad, f), F32),
            pltpu.VMEM((CHUNK,), jnp.int32),
            pltpu.VMEM((CHUNK,), jnp.int32),
            pltpu.VMEM((CHUNK, f), F32),
            pltpu.SemaphoreType.DMA,
        ],
    )


def _tc_scale_body(p0, p1, x, dinv_ref, xp_ref):
    deg = p0[...] + p1[...] + 1.0
    dinv = lax.rsqrt(deg)
    dinv_ref[...] = dinv
    xp_ref[...] = x[...] * dinv


def _tc_mlp_body(p0, p1, xp, dinv, w1, b1, w2, yp_ref):
    dv = dinv[...]
    agg = dv * (p0[...] + p1[...] + xp[...])
    h = jnp.maximum(jnp.dot(agg, w1[...], preferred_element_type=F32,
                            precision=lax.Precision.HIGHEST) + b1[...], 0.0)
    yp_ref[...] = jnp.dot(h, w2[...], preferred_element_type=F32,
                          precision=lax.Precision.HIGHEST) * dv


def _tc_out_body(q0, q1, yp, dinv, b2, out_ref):
    dv = dinv[...]
    out_ref[...] = jnp.maximum(dv * (q0[...] + q1[...] + yp[...]) + b2[...], 0.0)


def _row_spec(rb, w):
    return pl.BlockSpec((rb, w), lambda i: (i, 0))


def _full_spec(shape):
    return pl.BlockSpec(shape, lambda i: tuple(0 for _ in shape))


def _tc_scale(p0, p1, x, rb=2048):
    n_pad, f = x.shape
    return pl.pallas_call(
        _tc_scale_body,
        grid=(n_pad // rb,),
        in_specs=[_row_spec(rb, 1), _row_spec(rb, 1), _row_spec(rb, f)],
        out_specs=[_row_spec(rb, 1), _row_spec(rb, f)],
        out_shape=[jax.ShapeDtypeStruct((n_pad, 1), F32),
                   jax.ShapeDtypeStruct((n_pad, f), F32)],
    )(p0, p1, x)


def _tc_mlp(p0, p1, xp, dinv, w1, b1, w2, rb=2048):
    n_pad, f = xp.shape
    h = w1.shape[1]
    return pl.pallas_call(
        _tc_mlp_body,
        grid=(n_pad // rb,),
        in_specs=[_row_spec(rb, f), _row_spec(rb, f), _row_spec(rb, f),
                  _row_spec(rb, 1), _full_spec((f, h)), _full_spec((1, h)),
                  _full_spec((h, f))],
        out_specs=_row_spec(rb, f),
        out_shape=jax.ShapeDtypeStruct((n_pad, f), F32),
    )(p0, p1, xp, dinv, w1, b1, w2)


def _tc_out(q0, q1, yp, dinv, b2, rb=2048):
    n_pad, f = yp.shape
    return pl.pallas_call(
        _tc_out_body,
        grid=(n_pad // rb,),
        in_specs=[_row_spec(rb, f), _row_spec(rb, f), _row_spec(rb, f),
                  _row_spec(rb, 1), _full_spec((1, f))],
        out_specs=_row_spec(rb, f),
        out_shape=jax.ShapeDtypeStruct((n_pad, f), F32),
    )(q0, q1, yp, dinv, b2)


def kernel(x, edge_index, W1, b1, W2, b2):
    n, f = x.shape
    e = edge_index.shape[1]
    n_pad = ((n + 1 + 2047) // 2048) * 2048
    cpw = -(-e // (NW * CHUNK))
    e_pad = NW * cpw * CHUNK

    pad = jnp.full((e_pad - e,), n, jnp.int32)
    src = jnp.concatenate([edge_index[0].astype(jnp.int32), pad])
    dst = jnp.concatenate([edge_index[1].astype(jnp.int32), pad])
    x_pad = jnp.zeros((n_pad, f), F32).at[:n].set(x)
    zeros_row = jnp.zeros((n_pad,), F32)
    zeros_tab = jnp.zeros((n_pad, f), F32)

    deg_parts = _make_deg_kernel(n_pad, cpw)(dst, zeros_row)
    deg_cols = deg_parts.reshape(NC, n_pad, 1)
    dinv, xp = _tc_scale(deg_cols[0], deg_cols[1], x_pad)

    spmm = _make_spmm_kernel(n_pad, cpw, f)
    s1 = spmm(src, dst, xp, zeros_tab)
    yp = _tc_mlp(s1[0], s1[1], xp, dinv, W1, b1.reshape(1, -1), W2)
    s2 = spmm(src, dst, yp, zeros_tab)
    out = _tc_out(s2[0], s2[1], yp, dinv, b2.reshape(1, -1))
    return out[:n]
